# baseline (speedup 1.0000x reference)
import functools

import jax
import jax.numpy as jnp
import numpy as np
from jax import lax
from jax.experimental import pallas as pl
from jax.experimental.pallas import tpu as pltpu
from jax.experimental.pallas import tpu_sc as plsc

N = 10000
E = 320000
H = 64
P_MAX = 1.0
R_MIN = 1.0
N_PER_GRAPH = 1000
NC_CONS = 500

NCORE = 2
NSUB = 16
NW = NCORE * NSUB

CH = 128
SUP = 3
SE = SUP * CH
NSUPER = 27
EPT = NSUPER * SE
E_PAD = EPT * NW
ECH = E_PAD // CH
NPAD = 10240
RPT = NPAD // NSUB

T_PAD = 12288


def _make_edge_pass(D):
    mesh = plsc.VectorSubcoreMesh(core_axis_name="c", subcore_axis_name="s")

    @functools.partial(
        pl.kernel,
        out_type=jax.ShapeDtypeStruct((NCORE, NPAD, D), jnp.float32),
        mesh=mesh,
        compiler_params=pltpu.CompilerParams(use_tc_tiling_on_sc=False,
                                             needs_layout_passes=False),
        scratch_types=[
            pltpu.VMEM((3, SUP, CH), jnp.int32),
            pltpu.VMEM((3, SUP, CH), jnp.int32),
            pltpu.VMEM((3, SE), jnp.float32),
            pltpu.VMEM((3, SE, D), jnp.float32),
            pltpu.VMEM_SHARED((NPAD, D), jnp.float32),
            pltpu.SemaphoreType.DMA,
            pltpu.SemaphoreType.DMA,
            pltpu.SemaphoreType.DMA,
        ],
    )
    def edge_pass(z_hbm, src_hbm, dst_hbm, w_hbm, out_hbm,
                  sidx, didx, wbuf, rows, agg_sh, lsem, gsem, ssem):
        c = lax.axis_index("c")
        s = lax.axis_index("s")
        wid = s * NCORE + c
        cbase = wid * (NSUPER * SUP)

        def issue_loads(t, b):
            c0 = cbase + t * SUP
            pltpu.async_copy(src_hbm.at[pl.ds(c0, SUP)], sidx.at[b], lsem)
            pltpu.async_copy(dst_hbm.at[pl.ds(c0, SUP)], didx.at[b], lsem)
            pltpu.async_copy(w_hbm.at[wid * NSUPER + t], wbuf.at[b], lsem)

        def wait_loads(t, b):
            c0 = cbase + t * SUP
            pltpu.make_async_copy(src_hbm.at[pl.ds(c0, SUP)], sidx.at[b],
                                  lsem).wait()
            pltpu.make_async_copy(dst_hbm.at[pl.ds(c0, SUP)], didx.at[b],
                                  lsem).wait()
            pltpu.make_async_copy(w_hbm.at[wid * NSUPER + t], wbuf.at[b],
                                  lsem).wait()

        def issue_gathers(b):
            for j in range(SUP):
                pltpu.async_copy(z_hbm.at[sidx.at[b, j]],
                                 rows.at[b, pl.ds(j * CH, CH)], gsem)

        def wait_gathers(b):
            for j in range(SUP):
                pltpu.make_async_copy(z_hbm.at[sidx.at[b, j]],
                                      rows.at[b, pl.ds(j * CH, CH)],
                                      gsem).wait()

        def issue_scatters(b):
            for j in range(SUP):
                pltpu.async_copy(rows.at[b, pl.ds(j * CH, CH)],
                                 agg_sh.at[didx.at[b, j]], ssem, add=True)

        def drain_scatters(b):
            for j in range(SUP):
                pltpu.make_async_copy(rows.at[b, pl.ds(j * CH, CH)],
                                      agg_sh.at[didx.at[b, j]], ssem).wait()

        def zero_body(i, carry):
            for j in range(D // 16):
                rows[0, i, pl.ds(16 * j, 16)] = jnp.zeros((16,), jnp.float32)
            return carry
        lax.fori_loop(0, SE, zero_body, 0)
        row0 = s * RPT
        pltpu.sync_copy(rows.at[0], agg_sh.at[pl.ds(row0, SE)])
        pltpu.sync_copy(rows.at[0, pl.ds(0, RPT - SE)],
                        agg_sh.at[pl.ds(row0 + SE, RPT - SE)])
        plsc.subcore_barrier()

        def scale(b):
            bsplat = jnp.full((16,), b, jnp.int32)

            def scale_body(g, inner):
                for l in range(16):
                    e = 16 * g + l
                    wv = plsc.load_gather(
                        wbuf, [bsplat, jnp.full((16,), e, jnp.int32)])
                    for j in range(D // 16):
                        sl = pl.ds(16 * j, 16)
                        rows[b, e, sl] = rows[b, e, sl] * wv
                return inner
            lax.fori_loop(0, SE // 16, scale_body, 0)

        issue_loads(0, 0)
        wait_loads(0, 0)
        issue_gathers(0)

        def step(t, carry):
            b = t % 3
            pl.when(t >= 2)(lambda: drain_scatters((t - 2) % 3))
            pl.when(t + 1 < NSUPER)(lambda: issue_loads(t + 1, (t + 1) % 3))
            wait_gathers(b)
            scale(b)
            issue_scatters(b)

            def next_gathers():
                wait_loads(t + 1, (t + 1) % 3)
                issue_gathers((t + 1) % 3)
            pl.when(t + 1 < NSUPER)(next_gathers)
            return carry
        lax.fori_loop(0, NSUPER, step, 0)

        drain_scatters((NSUPER - 2) % 3)
        drain_scatters((NSUPER - 1) % 3)
        plsc.subcore_barrier()
        pltpu.sync_copy(agg_sh.at[pl.ds(row0, RPT)],
                        out_hbm.at[c, pl.ds(row0, RPT)])

    return edge_pass


_edge_pass_16 = _make_edge_pass(16)
_edge_pass_64 = _make_edge_pass(64)


def _dot(a, b):
    return jnp.dot(a, b, preferred_element_type=jnp.float32)


def _tc1_body(mu_ref, aggp_ref, ws0_ref, wn0_ref, b0_ref, ws1_ref,
              h1_ref, s1_ref):
    mu = mu_ref[...]
    B = mu.shape[0]
    r = lax.broadcasted_iota(jnp.int32, (B, 2), 0)
    lane = lax.broadcasted_iota(jnp.int32, (B, 2), 1)
    cons = jnp.where(r < NC_CONS, R_MIN, 0.0).astype(jnp.float32)
    x = jnp.where(lane == 0, jnp.broadcast_to(mu, (B, 2)), cons)
    agg = aggp_ref[0] + aggp_ref[1]
    h1 = jnp.maximum(
        _dot(x, ws0_ref[...]) + _dot(agg[:, 0:2], wn0_ref[...]) + b0_ref[...],
        0.0)
    h1_ref[...] = h1
    s1_ref[...] = _dot(h1, ws1_ref[...])


def _tc2_body(s1_ref, aggp_ref, wn1_ref, b1_ref, wout_ref, bout_ref, o16_ref):
    agg1 = aggp_ref[0] + aggp_ref[1]
    h2 = jnp.maximum(
        s1_ref[...] + _dot(agg1, wn1_ref[...]) + b1_ref[...], 0.0)
    logit = _dot(h2, wout_ref[...]) + bout_ref[...]
    o = P_MAX * (1.0 / (1.0 + jnp.exp(-logit)))
    o16_ref[...] = jnp.broadcast_to(o, o16_ref.shape)


_B = 1000


def _tc1(mu, agg0p, ws0, wn0, b0, ws1):
    return pl.pallas_call(
        _tc1_body,
        grid=(N // _B,),
        in_specs=[
            pl.BlockSpec((_B, 1), lambda i: (i, 0)),
            pl.BlockSpec((NCORE, _B, 16), lambda i: (0, i, 0)),
            pl.BlockSpec((2, H), lambda i: (0, 0)),
            pl.BlockSpec((2, H), lambda i: (0, 0)),
            pl.BlockSpec((1, H), lambda i: (0, 0)),
            pl.BlockSpec((H, H), lambda i: (0, 0)),
        ],
        out_specs=[
            pl.BlockSpec((_B, H), lambda i: (i, 0)),
            pl.BlockSpec((_B, H), lambda i: (i, 0)),
        ],
        out_shape=[
            jax.ShapeDtypeStruct((N, H), jnp.float32),
            jax.ShapeDtypeStruct((N, H), jnp.float32),
        ],
    )(mu, agg0p, ws0, wn0, b0, ws1)


def _tc2(s1, agg1p, wn1, b1, wout, bout):
    return pl.pallas_call(
        _tc2_body,
        grid=(N // _B,),
        in_specs=[
            pl.BlockSpec((_B, H), lambda i: (i, 0)),
            pl.BlockSpec((NCORE, _B, H), lambda i: (0, i, 0)),
            pl.BlockSpec((H, H), lambda i: (0, 0)),
            pl.BlockSpec((1, H), lambda i: (0, 0)),
            pl.BlockSpec((H, 1), lambda i: (0, 0)),
            pl.BlockSpec((1, 1), lambda i: (0, 0)),
        ],
        out_specs=pl.BlockSpec((_B, 16), lambda i: (i, 0)),
        out_shape=jax.ShapeDtypeStruct((N, 16), jnp.float32),
    )(s1, agg1p, wn1, b1, wout, bout)


mesh_g = plsc.VectorSubcoreMesh(core_axis_name="c", subcore_axis_name="s")


@functools.partial(
    pl.kernel,
    out_type=jax.ShapeDtypeStruct((T_PAD, 16), jnp.float32),
    mesh=mesh_g,
    compiler_params=pltpu.CompilerParams(use_tc_tiling_on_sc=False),
    scratch_types=[
        pltpu.VMEM((3, CH), jnp.int32),
        pltpu.VMEM((3, CH, 16), jnp.float32),
        pltpu.SemaphoreType.DMA,
        pltpu.SemaphoreType.DMA,
    ],
)
def _trans_gather(o16_hbm, t_hbm, out_hbm, idx_v, rows_v, lsem, gsem):
    c = lax.axis_index("c")
    s = lax.axis_index("s")
    wid = s * NCORE + c
    base = wid * (T_PAD // NW)
    nch = T_PAD // NW // CH
    for k in range(nch):
        pltpu.async_copy(t_hbm.at[pl.ds(base + k * CH, CH)], idx_v.at[k], lsem)
    for k in range(nch):
        pltpu.make_async_copy(t_hbm.at[pl.ds(base + k * CH, CH)],
                              idx_v.at[k], lsem).wait()
        pltpu.async_copy(o16_hbm.at[idx_v.at[k]], rows_v.at[k], gsem)
    for k in range(nch):
        pltpu.make_async_copy(o16_hbm.at[idx_v.at[k]], rows_v.at[k],
                              gsem).wait()
        pltpu.async_copy(rows_v.at[k], out_hbm.at[pl.ds(base + k * CH, CH)],
                         lsem)
    for k in range(nch):
        pltpu.make_async_copy(rows_v.at[k],
                              out_hbm.at[pl.ds(base + k * CH, CH)],
                              lsem).wait()


def kernel(mu, edge_index_l, edge_weight_l, transmitters_index,
           W_self0, W_nei0, b0, W_self1, W_nei1, b1, W_out, b_out):
    src = edge_index_l[0].astype(jnp.int32)
    dst = edge_index_l[1].astype(jnp.int32)
    w = edge_weight_l.astype(jnp.float32)
    pad = E_PAD - E
    srcp = jnp.pad(src, (0, pad)).reshape(ECH, CH)
    dstp = jnp.pad(dst, (0, pad)).reshape(ECH, CH)
    wp = jnp.pad(w, (0, pad)).reshape(NW * NSUPER, SE)

    cons = jnp.where((jnp.arange(N) % N_PER_GRAPH) < NC_CONS, R_MIN, 0.0)
    x16 = jnp.concatenate(
        [mu, cons[:, None].astype(jnp.float32), jnp.zeros((N, 14), jnp.float32)],
        axis=1)

    agg0p = _edge_pass_16(x16, srcp, dstp, wp)
    h1, s1 = _tc1(mu, agg0p, W_self0, W_nei0, b0.reshape(1, H), W_self1)
    agg1p = _edge_pass_64(h1, srcp, dstp, wp)
    o16 = _tc2(s1, agg1p, W_nei1, b1.reshape(1, H), W_out,
               b_out.reshape(1, 1))
    tp = jnp.pad(transmitters_index.astype(jnp.int32), (0, T_PAD - N))
    pg = _trans_gather(o16, tp)
    return pg[:N, :1]


# speedup vs baseline: 1.3802x; 1.3802x over previous
import functools

import jax
import jax.numpy as jnp
import numpy as np
from jax import lax
from jax.experimental import pallas as pl
from jax.experimental.pallas import tpu as pltpu
from jax.experimental.pallas import tpu_sc as plsc

N = 10000
E = 320000
H = 64
P_MAX = 1.0
R_MIN = 1.0
N_PER_GRAPH = 1000
NC_CONS = 500

NCORE = 2
NSUB = 16
NW = NCORE * NSUB

CH = 128
SUP = 3
SE = SUP * CH
NSUPER = 27
EPT = NSUPER * SE
E_PAD = EPT * NW
ECH = E_PAD // CH
NPAD = 10240
RPT = NPAD // NSUB

T_PAD = 12288


def _make_edge_pass(D):
    mesh = plsc.VectorSubcoreMesh(core_axis_name="c", subcore_axis_name="s")

    @functools.partial(
        pl.kernel,
        out_type=jax.ShapeDtypeStruct((NCORE, NPAD, D), jnp.float32),
        mesh=mesh,
        compiler_params=pltpu.CompilerParams(use_tc_tiling_on_sc=False),
        scratch_types=[
            pltpu.VMEM((3, SUP, CH), jnp.int32),
            pltpu.VMEM((3, SUP, CH), jnp.int32),
            pltpu.VMEM((3, SE), jnp.float32),
            pltpu.VMEM((3, SE, D), jnp.float32),
            pltpu.VMEM_SHARED((NPAD, D), jnp.float32),
            pltpu.SemaphoreType.DMA,
            pltpu.SemaphoreType.DMA,
            pltpu.SemaphoreType.DMA,
        ],
    )
    def edge_pass(z_hbm, src_hbm, dst_hbm, w_hbm, out_hbm,
                  sidx, didx, wbuf, rows, agg_sh, lsem, gsem, ssem):
        c = lax.axis_index("c")
        s = lax.axis_index("s")
        wid = s * NCORE + c
        cbase = wid * (NSUPER * SUP)

        def issue_loads(t, b):
            c0 = cbase + t * SUP
            pltpu.async_copy(src_hbm.at[pl.ds(c0, SUP)], sidx.at[b], lsem)
            pltpu.async_copy(dst_hbm.at[pl.ds(c0, SUP)], didx.at[b], lsem)
            pltpu.async_copy(w_hbm.at[wid * NSUPER + t], wbuf.at[b], lsem)

        def wait_loads(t, b):
            c0 = cbase + t * SUP
            pltpu.make_async_copy(src_hbm.at[pl.ds(c0, SUP)], sidx.at[b],
                                  lsem).wait()
            pltpu.make_async_copy(dst_hbm.at[pl.ds(c0, SUP)], didx.at[b],
                                  lsem).wait()
            pltpu.make_async_copy(w_hbm.at[wid * NSUPER + t], wbuf.at[b],
                                  lsem).wait()

        def issue_gathers(b):
            for j in range(SUP):
                pltpu.async_copy(z_hbm.at[sidx.at[b, j]],
                                 rows.at[b, pl.ds(j * CH, CH)], gsem)

        def wait_gathers(b):
            for j in range(SUP):
                pltpu.make_async_copy(z_hbm.at[sidx.at[b, j]],
                                      rows.at[b, pl.ds(j * CH, CH)],
                                      gsem).wait()

        def issue_scatters(b):
            for j in range(SUP):
                pltpu.async_copy(rows.at[b, pl.ds(j * CH, CH)],
                                 agg_sh.at[didx.at[b, j]], ssem, add=True)

        def drain_scatters(b):
            for j in range(SUP):
                pltpu.make_async_copy(rows.at[b, pl.ds(j * CH, CH)],
                                      agg_sh.at[didx.at[b, j]], ssem).wait()

        def zero_body(i, carry):
            for j in range(D // 16):
                rows[0, i, pl.ds(16 * j, 16)] = jnp.zeros((16,), jnp.float32)
            return carry
        lax.fori_loop(0, SE, zero_body, 0)
        row0 = s * RPT
        pltpu.sync_copy(rows.at[0], agg_sh.at[pl.ds(row0, SE)])
        pltpu.sync_copy(rows.at[0, pl.ds(0, RPT - SE)],
                        agg_sh.at[pl.ds(row0 + SE, RPT - SE)])
        plsc.subcore_barrier()

        def scale(b):
            def scale_body(g, inner):
                w16 = wbuf[b, pl.ds(16 * g, 16)]
                for l in range(16):
                    wv = w16[l]
                    e = 16 * g + l
                    for j in range(D // 16):
                        sl = pl.ds(16 * j, 16)
                        rows[b, e, sl] = rows[b, e, sl] * wv
                return inner
            lax.fori_loop(0, SE // 16, scale_body, 0)

        issue_loads(0, 0)
        wait_loads(0, 0)
        issue_gathers(0)

        def step(t, carry):
            b = t % 3
            pl.when(t >= 2)(lambda: drain_scatters((t - 2) % 3))
            pl.when(t + 1 < NSUPER)(lambda: issue_loads(t + 1, (t + 1) % 3))
            wait_gathers(b)

            def next_gathers():
                wait_loads(t + 1, (t + 1) % 3)
                issue_gathers((t + 1) % 3)
            pl.when(t + 1 < NSUPER)(next_gathers)
            scale(b)
            issue_scatters(b)
            return carry
        lax.fori_loop(0, NSUPER, step, 0)

        drain_scatters((NSUPER - 2) % 3)
        drain_scatters((NSUPER - 1) % 3)
        plsc.subcore_barrier()
        pltpu.sync_copy(agg_sh.at[pl.ds(row0, RPT)],
                        out_hbm.at[c, pl.ds(row0, RPT)])

    return edge_pass


_edge_pass_16 = _make_edge_pass(16)
_edge_pass_64 = _make_edge_pass(64)


def _dot(a, b):
    return jnp.dot(a, b, preferred_element_type=jnp.float32)


def _tc1_body(mu_ref, aggp_ref, ws0_ref, wn0_ref, b0_ref, ws1_ref,
              h1_ref, s1_ref):
    mu = mu_ref[...]
    B = mu.shape[0]
    r = lax.broadcasted_iota(jnp.int32, (B, 2), 0)
    lane = lax.broadcasted_iota(jnp.int32, (B, 2), 1)
    cons = jnp.where(r < NC_CONS, R_MIN, 0.0).astype(jnp.float32)
    x = jnp.where(lane == 0, jnp.broadcast_to(mu, (B, 2)), cons)
    agg = aggp_ref[0] + aggp_ref[1]
    h1 = jnp.maximum(
        _dot(x, ws0_ref[...]) + _dot(agg[:, 0:2], wn0_ref[...]) + b0_ref[...],
        0.0)
    h1_ref[...] = h1
    s1_ref[...] = _dot(h1, ws1_ref[...])


def _tc2_body(s1_ref, aggp_ref, wn1_ref, b1_ref, wout_ref, bout_ref, o16_ref):
    agg1 = aggp_ref[0] + aggp_ref[1]
    h2 = jnp.maximum(
        s1_ref[...] + _dot(agg1, wn1_ref[...]) + b1_ref[...], 0.0)
    logit = _dot(h2, wout_ref[...]) + bout_ref[...]
    o = P_MAX * (1.0 / (1.0 + jnp.exp(-logit)))
    o16_ref[...] = jnp.broadcast_to(o, o16_ref.shape)


_B = 1000


def _tc1(mu, agg0p, ws0, wn0, b0, ws1):
    return pl.pallas_call(
        _tc1_body,
        grid=(N // _B,),
        in_specs=[
            pl.BlockSpec((_B, 1), lambda i: (i, 0)),
            pl.BlockSpec((NCORE, _B, 16), lambda i: (0, i, 0)),
            pl.BlockSpec((2, H), lambda i: (0, 0)),
            pl.BlockSpec((2, H), lambda i: (0, 0)),
            pl.BlockSpec((1, H), lambda i: (0, 0)),
            pl.BlockSpec((H, H), lambda i: (0, 0)),
        ],
        out_specs=[
            pl.BlockSpec((_B, H), lambda i: (i, 0)),
            pl.BlockSpec((_B, H), lambda i: (i, 0)),
        ],
        out_shape=[
            jax.ShapeDtypeStruct((N, H), jnp.float32),
            jax.ShapeDtypeStruct((N, H), jnp.float32),
        ],
    )(mu, agg0p, ws0, wn0, b0, ws1)


def _tc2(s1, agg1p, wn1, b1, wout, bout):
    return pl.pallas_call(
        _tc2_body,
        grid=(N // _B,),
        in_specs=[
            pl.BlockSpec((_B, H), lambda i: (i, 0)),
            pl.BlockSpec((NCORE, _B, H), lambda i: (0, i, 0)),
            pl.BlockSpec((H, H), lambda i: (0, 0)),
            pl.BlockSpec((1, H), lambda i: (0, 0)),
            pl.BlockSpec((H, 1), lambda i: (0, 0)),
            pl.BlockSpec((1, 1), lambda i: (0, 0)),
        ],
        out_specs=pl.BlockSpec((_B, 16), lambda i: (i, 0)),
        out_shape=jax.ShapeDtypeStruct((N, 16), jnp.float32),
    )(s1, agg1p, wn1, b1, wout, bout)


mesh_g = plsc.VectorSubcoreMesh(core_axis_name="c", subcore_axis_name="s")


@functools.partial(
    pl.kernel,
    out_type=jax.ShapeDtypeStruct((T_PAD, 16), jnp.float32),
    mesh=mesh_g,
    compiler_params=pltpu.CompilerParams(use_tc_tiling_on_sc=False),
    scratch_types=[
        pltpu.VMEM((3, CH), jnp.int32),
        pltpu.VMEM((3, CH, 16), jnp.float32),
        pltpu.SemaphoreType.DMA,
        pltpu.SemaphoreType.DMA,
    ],
)
def _trans_gather(o16_hbm, t_hbm, out_hbm, idx_v, rows_v, lsem, gsem):
    c = lax.axis_index("c")
    s = lax.axis_index("s")
    wid = s * NCORE + c
    base = wid * (T_PAD // NW)
    nch = T_PAD // NW // CH
    for k in range(nch):
        pltpu.async_copy(t_hbm.at[pl.ds(base + k * CH, CH)], idx_v.at[k], lsem)
    for k in range(nch):
        pltpu.make_async_copy(t_hbm.at[pl.ds(base + k * CH, CH)],
                              idx_v.at[k], lsem).wait()
        pltpu.async_copy(o16_hbm.at[idx_v.at[k]], rows_v.at[k], gsem)
    for k in range(nch):
        pltpu.make_async_copy(o16_hbm.at[idx_v.at[k]], rows_v.at[k],
                              gsem).wait()
        pltpu.async_copy(rows_v.at[k], out_hbm.at[pl.ds(base + k * CH, CH)],
                         lsem)
    for k in range(nch):
        pltpu.make_async_copy(rows_v.at[k],
                              out_hbm.at[pl.ds(base + k * CH, CH)],
                              lsem).wait()


def kernel(mu, edge_index_l, edge_weight_l, transmitters_index,
           W_self0, W_nei0, b0, W_self1, W_nei1, b1, W_out, b_out):
    src = edge_index_l[0].astype(jnp.int32)
    dst = edge_index_l[1].astype(jnp.int32)
    w = edge_weight_l.astype(jnp.float32)
    pad = E_PAD - E
    srcp = jnp.pad(src, (0, pad)).reshape(ECH, CH)
    dstp = jnp.pad(dst, (0, pad)).reshape(ECH, CH)
    wp = jnp.pad(w, (0, pad)).reshape(NW * NSUPER, SE)

    cons = jnp.where((jnp.arange(N) % N_PER_GRAPH) < NC_CONS, R_MIN, 0.0)
    x16 = jnp.concatenate(
        [mu, cons[:, None].astype(jnp.float32), jnp.zeros((N, 14), jnp.float32)],
        axis=1)

    agg0p = _edge_pass_16(x16, srcp, dstp, wp)
    h1, s1 = _tc1(mu, agg0p, W_self0, W_nei0, b0.reshape(1, H), W_self1)
    agg1p = _edge_pass_64(h1, srcp, dstp, wp)
    o16 = _tc2(s1, agg1p, W_nei1, b1.reshape(1, H), W_out,
               b_out.reshape(1, 1))
    tp = jnp.pad(transmitters_index.astype(jnp.int32), (0, T_PAD - N))
    pg = _trans_gather(o16, tp)
    return pg[:N, :1]
